# baseline (speedup 1.0000x reference)
import jax
import jax.numpy as jnp
from jax.experimental import pallas as pl
from jax.experimental.pallas import tpu as pltpu


def _gate_scale(H):
    lane = jax.lax.broadcasted_iota(jnp.int32, (1, 4 * H), 1)
    return jnp.where((lane < 2 * H) | (lane >= 3 * H), 0.5, 1.0)


def _lstm_steps(whb_sc, gx_sc, hout_ref, h, c, base):
    TC = gx_sc.shape[0]
    H = h.shape[1]
    for t in range(TC):
        g = jnp.dot(h, whb_sc[...],
                    preferred_element_type=jnp.float32) + gx_sc[t]
        t_if = jnp.tanh(g[:, :2 * H])
        t_g = jnp.tanh(g[:, 2 * H:3 * H])
        t_o = jnp.tanh(g[:, 3 * H:])
        c = 0.5 * (c + t_if[:, H:] * c + (1.0 + t_if[:, :H]) * t_g)
        h = (0.5 * (1.0 + t_o)) * jnp.tanh(c)
        hout_ref[base + t] = h
    return h, c


def _l0_kernel(idxa_ref, idxb_ref, idxc_ref, emb_ref, wx_ref, wh_ref, b_ref,
               h0_ref, c0_ref, hout_ref, cfin_ref,
               ew_sc, gxa_sc, gxb_sc, h_sc, c_sc, whb_sc):
    TC = gxa_sc.shape[0]
    H = h_sc.shape[1]
    V = emb_ref.shape[0]
    scale = _gate_scale(H)
    bias = b_ref[0] * scale

    def gx_from(idx_ref, out_sc):
        idx = idx_ref[0]
        iota_v = jax.lax.broadcasted_iota(jnp.int32, (V, idx.shape[1]), 0)
        oh_t = (iota_v == idx).astype(jnp.float32)
        gx = jax.lax.dot_general(
            oh_t, ew_sc[...],
            dimension_numbers=(((0,), (0,)), ((), ())),
            preferred_element_type=jnp.float32) + bias
        out_sc[...] = gx.reshape(out_sc.shape)

    @pl.when(pl.program_id(0) == 0)
    def _():
        ew_sc[...] = jnp.dot(emb_ref[...], wx_ref[0],
                             preferred_element_type=jnp.float32) * scale
        whb_sc[...] = (wh_ref[0] * scale).astype(jnp.bfloat16)
        h_sc[...] = h0_ref[0]
        c_sc[...] = c0_ref[0]
        gx_from(idxa_ref, gxa_sc)

    h, c = h_sc[...], c_sc[...]
    gx_from(idxb_ref, gxb_sc)
    h, c = _lstm_steps(whb_sc, gxa_sc, hout_ref, h, c, 0)
    gx_from(idxc_ref, gxa_sc)
    h, c = _lstm_steps(whb_sc, gxb_sc, hout_ref, h, c, TC)
    h_sc[...] = h
    c_sc[...] = c
    cfin_ref[...] = c


def _l1_kernel(hina_ref, hinb_ref, hinc_ref, wx_ref, wh_ref, b_ref,
               h0_ref, c0_ref, hout_ref, cfin_ref,
               gxa_sc, gxb_sc, h_sc, c_sc, whb_sc, wxb_sc):
    TC, B, H = hina_ref.shape
    scale = _gate_scale(H)
    bias = b_ref[0] * scale

    def gx_from(hin_ref, out_sc):
        x = hin_ref[...].reshape(TC * B, H)
        gx = jnp.dot(x, wxb_sc[...],
                     preferred_element_type=jnp.float32) + bias
        out_sc[...] = gx.reshape(out_sc.shape)

    @pl.when(pl.program_id(0) == 0)
    def _():
        whb_sc[...] = (wh_ref[0] * scale).astype(jnp.bfloat16)
        wxb_sc[...] = (wx_ref[0] * scale).astype(jnp.bfloat16)
        h_sc[...] = h0_ref[0]
        c_sc[...] = c0_ref[0]
        gx_from(hina_ref, gxa_sc)

    h, c = h_sc[...], c_sc[...]
    gx_from(hinb_ref, gxb_sc)
    h, c = _lstm_steps(whb_sc, gxa_sc, hout_ref, h, c, 0)
    gx_from(hinc_ref, gxa_sc)
    h, c = _lstm_steps(whb_sc, gxb_sc, hout_ref, h, c, TC)
    h_sc[...] = h
    c_sc[...] = c
    cfin_ref[...] = c


def _dec_kernel(x_ref, w_ref, b_ref, o_ref):
    o_ref[...] = jnp.dot(x_ref[...], w_ref[0],
                         preferred_element_type=jnp.float32) + b_ref[0]


def kernel(idx_seq, emb, w_all, b_all, h0, c0):
    T, B = idx_seq.shape
    V, H = emb.shape
    G = 4 * H
    O = 256
    TB = T * B
    TC = 16 if T % 32 == 0 else T // 2
    NT = T // TC
    NP = NT // 2

    idx_r = idx_seq.astype(jnp.int32).reshape(NT, 1, TC * B)

    def layer_specs(l):
        return [
            pl.BlockSpec((1, H, G), lambda j, l=l: (l, 0, 0)),
            pl.BlockSpec((1, H, G), lambda j, l=l: (l, 1, 0)),
            pl.BlockSpec((1, 1, G), lambda j, l=l: (l, 0, 0)),
            pl.BlockSpec((1, B, H), lambda j, l=l: (l, 0, 0)),
            pl.BlockSpec((1, B, H), lambda j, l=l: (l, 0, 0)),
        ]

    out_specs = [
        pl.BlockSpec((2 * TC, B, H), lambda j: (j, 0, 0)),
        pl.BlockSpec((B, H), lambda j: (0, 0)),
    ]
    out_shape = [jax.ShapeDtypeStruct((T, B, H), jnp.float32),
                 jax.ShapeDtypeStruct((B, H), jnp.float32)]
    state_scratch = [pltpu.VMEM((TC, B, G), jnp.float32),
                     pltpu.VMEM((TC, B, G), jnp.float32),
                     pltpu.VMEM((B, H), jnp.float32),
                     pltpu.VMEM((B, H), jnp.float32),
                     pltpu.VMEM((H, G), jnp.bfloat16)]
    seq_sem = pltpu.CompilerParams(dimension_semantics=("arbitrary",))
    last = NT - 1

    idx_specs = [
        pl.BlockSpec((1, 1, TC * B), lambda j: (2 * j, 0, 0)),
        pl.BlockSpec((1, 1, TC * B), lambda j: (2 * j + 1, 0, 0)),
        pl.BlockSpec((1, 1, TC * B), lambda j, n=last: (jnp.minimum(2 * j + 2, n), 0, 0)),
    ]
    h_all0, c_fin0 = pl.pallas_call(
        _l0_kernel,
        grid=(NP,),
        in_specs=idx_specs
                 + [pl.BlockSpec((V, H), lambda j: (0, 0))] + layer_specs(0),
        out_specs=out_specs,
        out_shape=out_shape,
        scratch_shapes=[pltpu.VMEM((V, G), jnp.float32)] + state_scratch,
        compiler_params=seq_sem,
    )(idx_r, idx_r, idx_r, emb, w_all, w_all, b_all, h0, c0)

    hin_specs = [
        pl.BlockSpec((TC, B, H), lambda j: (2 * j, 0, 0)),
        pl.BlockSpec((TC, B, H), lambda j: (2 * j + 1, 0, 0)),
        pl.BlockSpec((TC, B, H), lambda j, n=last: (jnp.minimum(2 * j + 2, n), 0, 0)),
    ]
    h_all1, c_fin1 = pl.pallas_call(
        _l1_kernel,
        grid=(NP,),
        in_specs=hin_specs + layer_specs(1),
        out_specs=out_specs,
        out_shape=out_shape,
        scratch_shapes=state_scratch + [pltpu.VMEM((H, G), jnp.bfloat16)],
        compiler_params=seq_sem,
    )(h_all0, h_all0, h_all0, w_all, w_all, b_all, h0, c0)

    MBd = TB // 4
    logits = pl.pallas_call(
        _dec_kernel,
        grid=(2, 2),
        in_specs=[
            pl.BlockSpec((MBd, H), lambda bi, j: (bi * 2 + j, 0)),
            pl.BlockSpec((1, H, O), lambda bi, j: (2, 0, 0)),
            pl.BlockSpec((1, 1, O), lambda bi, j: (2, 0, 0)),
        ],
        out_specs=pl.BlockSpec((MBd, O), lambda bi, j: (bi * 2 + j, 0)),
        out_shape=jax.ShapeDtypeStruct((TB, O), jnp.float32),
        compiler_params=pltpu.CompilerParams(
            dimension_semantics=("parallel", "arbitrary")),
    )(h_all1.reshape(TB, H), w_all, b_all)

    h_n = jnp.stack([h_all0[T - 1], h_all1[T - 1]])
    c_n = jnp.stack([c_fin0, c_fin1])
    return logits.reshape(T, B, O), (h_n, c_n)
